# baseline (speedup 1.0000x reference)
import jax
import jax.numpy as jnp
from jax.experimental import pallas as pl
from jax.experimental.pallas import tpu as pltpu

_B = 128
_N = 100000
_K = 64
_R = 16
_S = 32
_G = _N // _S
_ONE_BITS = 0x3F800000


def _loss_kernel(x_ref, out_ref):
    x = x_ref[...]
    bits = jax.lax.bitcast_convert_type(x, jnp.int32)

    l = jnp.log(1.0 - x)
    s_all = jnp.sum(l)
    m = jnp.max(x, axis=1)
    mbits = jax.lax.bitcast_convert_type(m, jnp.int32)

    ones_mat = jnp.ones((_G, 128), jnp.float32)
    lo = jnp.zeros((_R, 1), jnp.int32)
    hi = jnp.full((_R, 1), 1 << 30, jnp.int32)
    kf = jnp.float32(_K)
    for _ in range(15):
        q = (hi - lo) >> 2
        m1 = lo + q
        m2 = lo + 2 * q
        m3 = lo + 3 * q
        c1 = jax.lax.dot((mbits >= m1).astype(jnp.float32), ones_mat)[:, :1]
        c2 = jax.lax.dot((mbits >= m2).astype(jnp.float32), ones_mat)[:, :1]
        c3 = jax.lax.dot((mbits >= m3).astype(jnp.float32), ones_mat)[:, :1]
        ok1, ok2, ok3 = c1 >= kf, c2 >= kf, c3 >= kf
        lo = jnp.where(ok3, m3, jnp.where(ok2, m2, jnp.where(ok1, m1, lo)))
        hi = jnp.where(ok3, hi, jnp.where(ok2, m3, jnp.where(ok1, m2, m1)))
    gstar = lo[:, :, None]

    cand = bits >= gstar
    c0 = jnp.sum(cand.astype(jnp.int32), axis=(1, 2))[:, None, None]
    sum_cand = jnp.sum(jnp.where(cand, l, 0.0), axis=(1, 2))

    def cond(carry):
        _, c_rem, _ = carry
        return jnp.any(c_rem > _K)

    def body(carry):
        b, c_rem, acc = carry
        active = c_rem > _K
        inc = bits >= b
        mn = jnp.min(jnp.where(inc, x, 2.0), axis=(1, 2))[:, None, None]
        n_eq = jnp.sum((inc & (x == mn)).astype(jnp.int32),
                       axis=(1, 2))[:, None, None]
        rem_all = active & (c_rem - n_eq >= _K)
        rem_part = active & ~rem_all
        lm = jnp.log(1.0 - jnp.where(active, mn, 0.0))
        acc = acc + jnp.where(
            rem_all, n_eq.astype(jnp.float32) * lm,
            jnp.where(rem_part, (c_rem - _K).astype(jnp.float32) * lm, 0.0))
        c_rem = jnp.where(rem_all, c_rem - n_eq,
                          jnp.where(rem_part, _K, c_rem))
        mn_b = jax.lax.bitcast_convert_type(mn, jnp.int32)
        b = jnp.where(rem_all, mn_b + 1, b)
        return b, c_rem, acc

    _, _, acc = jax.lax.while_loop(
        cond, body, (gstar, c0, jnp.zeros((_R, 1, 1), jnp.float32)))

    t_sum = jnp.sum(sum_cand) - jnp.sum(acc)
    partial = s_all - t_sum

    @pl.when(pl.program_id(0) == 0)
    def _():
        out_ref[0, 0] = 0.0

    out_ref[0, 0] += -partial / jnp.float32(_N)


def kernel(top_c, output):
    x3 = output.reshape(_B, _S, _G)
    out = pl.pallas_call(
        _loss_kernel,
        grid=(_B // _R,),
        in_specs=[pl.BlockSpec((_R, _S, _G), lambda i: (i, 0, 0))],
        out_specs=pl.BlockSpec(
            (1, 1), lambda i: (0, 0), memory_space=pltpu.SMEM
        ),
        out_shape=jax.ShapeDtypeStruct((1, 1), jnp.float32),
    )(x3)
    loss = out[0, 0]
    return loss + 0.0 * jnp.asarray(top_c, dtype=loss.dtype)


# speedup vs baseline: 1.6216x; 1.6216x over previous
import jax
import jax.numpy as jnp
from jax.experimental import pallas as pl
from jax.experimental.pallas import tpu as pltpu

_B = 128
_N = 100000
_K = 64
_R = 16
_S = 32
_G = _N // _S
_C2 = 2 * _G
_ONE_BITS = 0x3F800000


def _k1(x_ref, s_ref, c_ref):
    x = x_ref[...]
    l = jnp.log(1.0 - x)

    @pl.when(pl.program_id(0) == 0)
    def _():
        s_ref[0, 0] = 0.0

    s_ref[0, 0] += jnp.sum(l)

    a = jnp.maximum(x[:, 0:16], x[:, 16:32])
    b = jnp.minimum(x[:, 0:16], x[:, 16:32])
    for h in (8, 4, 2, 1):
        a1, a2 = a[:, 0:h], a[:, h : 2 * h]
        b1, b2 = b[:, 0:h], b[:, h : 2 * h]
        a = jnp.maximum(a1, a2)
        b = jnp.maximum(jnp.minimum(a1, a2), jnp.maximum(b1, b2))
    c_ref[...] = jnp.concatenate([a[:, 0], b[:, 0]], axis=-1)


def _k2(x_ref, c_ref, s_ref, out_ref, tb_s, cc_s, tf_s):
    p = pl.program_id(0)

    @pl.when(p == 0)
    def _():
        cv = c_ref[...]
        cb = jax.lax.bitcast_convert_type(cv, jnp.int32)
        lo0 = jnp.zeros((_B, 1), jnp.int32)
        hi0 = jnp.full((_B, 1), _ONE_BITS, jnp.int32)

        def bis(_, carry):
            lo, hi = carry
            mid = (lo + hi) // 2
            cnt = jnp.sum((cb >= mid).astype(jnp.int32), axis=1,
                          keepdims=True)
            take = cnt >= _K
            return jnp.where(take, mid, lo), jnp.where(take, hi, mid)

        tb, _ = jax.lax.fori_loop(0, 30, bis, (lo0, hi0))
        t = jax.lax.bitcast_convert_type(tb, jnp.float32)
        lc = jnp.log(1.0 - cv)
        gt = cb > tb
        ge = cb >= tb
        cnt_gt = jnp.sum(gt.astype(jnp.int32), axis=1, keepdims=True)
        sum_gt = jnp.sum(jnp.where(gt, lc, 0.0), axis=1, keepdims=True)
        tf = sum_gt + (jnp.float32(_K) - cnt_gt.astype(jnp.float32)) \
            * jnp.log(1.0 - t)
        tb_s[...] = tb
        cc_s[...] = jnp.sum(ge.astype(jnp.int32), axis=1, keepdims=True)
        tf_s[...] = tf
        out_ref[0, 0] = -s_ref[0, 0] / jnp.float32(_N)

    x = x_ref[...]
    bits = jax.lax.bitcast_convert_type(x, jnp.int32)
    tb_r = tb_s[pl.ds(p * _R, _R), :]
    cc_r = cc_s[pl.ds(p * _R, _R), :]
    tf_r = tf_s[pl.ds(p * _R, _R), :]

    tb3 = tb_r[:, :, None]
    cand = bits >= tb3
    c0 = jnp.sum(cand.astype(jnp.int32), axis=(1, 2))[:, None]

    def fast():
        return tf_r

    def slow():
        l = jnp.log(1.0 - x)
        sum_cand = jnp.sum(jnp.where(cand, l, 0.0), axis=(1, 2))[:, None]

        def cond(carry):
            _, c_rem, _ = carry
            return jnp.any(c_rem > _K)

        def body(carry):
            b, c_rem, acc = carry
            active = c_rem > _K
            inc = bits >= b
            mn = jnp.min(jnp.where(inc, x, 2.0), axis=(1, 2))[:, None, None]
            n_eq = jnp.sum((inc & (x == mn)).astype(jnp.int32),
                           axis=(1, 2))[:, None, None]
            rem_all = active & (c_rem - n_eq >= _K)
            rem_part = active & ~rem_all
            lm = jnp.log(1.0 - jnp.where(active, mn, 0.0))
            acc = acc + jnp.where(
                rem_all, n_eq.astype(jnp.float32) * lm,
                jnp.where(rem_part,
                          (c_rem - _K).astype(jnp.float32) * lm, 0.0))
            c_rem = jnp.where(rem_all, c_rem - n_eq,
                              jnp.where(rem_part, _K, c_rem))
            mn_b = jax.lax.bitcast_convert_type(mn, jnp.int32)
            b = jnp.where(rem_all, mn_b + 1, b)
            return b, c_rem, acc

        _, _, acc = jax.lax.while_loop(
            cond, body,
            (tb3, c0[:, :, None], jnp.zeros((_R, 1, 1), jnp.float32)))
        return sum_cand - acc[:, :, 0]

    t_rows = jax.lax.cond(jnp.any(c0 != cc_r), slow, fast)
    out_ref[0, 0] += jnp.sum(t_rows) / jnp.float32(_N)


def kernel(top_c, output):
    x3 = output.reshape(_B, _S, _G)
    s_part, c_arr = pl.pallas_call(
        _k1,
        grid=(_B // _R,),
        in_specs=[pl.BlockSpec((_R, _S, _G), lambda i: (i, 0, 0))],
        out_specs=[
            pl.BlockSpec((1, 1), lambda i: (0, 0), memory_space=pltpu.SMEM),
            pl.BlockSpec((_R, _C2), lambda i: (i, 0)),
        ],
        out_shape=[
            jax.ShapeDtypeStruct((1, 1), jnp.float32),
            jax.ShapeDtypeStruct((_B, _C2), jnp.float32),
        ],
    )(x3)
    out = pl.pallas_call(
        _k2,
        grid=(_B // _R,),
        in_specs=[
            pl.BlockSpec((_R, _S, _G), lambda i: (i, 0, 0)),
            pl.BlockSpec((_B, _C2), lambda i: (0, 0)),
            pl.BlockSpec((1, 1), lambda i: (0, 0), memory_space=pltpu.SMEM),
        ],
        out_specs=pl.BlockSpec(
            (1, 1), lambda i: (0, 0), memory_space=pltpu.SMEM
        ),
        out_shape=jax.ShapeDtypeStruct((1, 1), jnp.float32),
        scratch_shapes=[
            pltpu.VMEM((_B, 1), jnp.int32),
            pltpu.VMEM((_B, 1), jnp.int32),
            pltpu.VMEM((_B, 1), jnp.float32),
        ],
    )(x3, c_arr, s_part)
    loss = out[0, 0]
    return loss + 0.0 * jnp.asarray(top_c, dtype=loss.dtype)
